# baseline (speedup 1.0000x reference)
import functools

import jax
import jax.numpy as jnp
from jax import lax
from jax.experimental import pallas as pl
from jax.experimental.pallas import tpu as pltpu
from jax.experimental.pallas import tpu_sc as plsc

_NC = 2
_NS = 16
_NW = _NC * _NS
_L = 16


def _rsqrt16(s):
    i = lax.bitcast_convert_type(s, jnp.int32)
    i = jnp.int32(0x5F3759DF) - (i >> 1)
    y = lax.bitcast_convert_type(i, jnp.float32)
    for _ in range(3):
        y = y * (1.5 - 0.5 * s * y * y)
    return y


def _make_sc_kernel(B, D, chunk):
    b_per_w = B // _NW
    n_chunks = b_per_w // chunk
    n_idx_sub = chunk // 128
    groups_per_chunk = chunk // _L
    mesh = plsc.VectorSubcoreMesh(core_axis_name="c", subcore_axis_name="s")

    @functools.partial(
        pl.kernel,
        mesh=mesh,
        compiler_params=pltpu.CompilerParams(
            needs_layout_passes=False,
            use_tc_tiling_on_sc=False,
        ),
        out_type=jax.ShapeDtypeStruct((B, D), jnp.float32),
        scratch_types=[
            pltpu.VMEM((b_per_w // 128, 128), jnp.int32),
            pltpu.VMEM((chunk, D), jnp.float32),
            pltpu.VMEM((chunk, D), jnp.float32),
            pltpu.VMEM((chunk, D), jnp.float32),
            pltpu.SemaphoreType.DMA,
        ],
    )
    def k(trans_hbm, idx_hbm, emb_hbm, out_hbm, idx_v, trans_v, emb_v, out_v, sem):
        wid = lax.axis_index("s") * _NC + lax.axis_index("c")
        base = wid * b_per_w
        iota16 = lax.iota(jnp.int32, _L)

        for r in range(b_per_w // 128):
            pltpu.sync_copy(idx_hbm.at[pl.ds(base + r * 128, 128)], idx_v.at[r])

        for c in range(n_chunks):
            row0 = base + c * chunk
            copies = []
            for r in range(n_idx_sub):
                copies.append(
                    pltpu.async_copy(
                        trans_hbm.at[idx_v.at[c * n_idx_sub + r]],
                        trans_v.at[pl.ds(r * 128, 128)],
                        sem,
                    )
                )
            copies.append(pltpu.async_copy(emb_hbm.at[pl.ds(row0, chunk)], emb_v, sem))
            for cp in copies:
                cp.wait()

            def group(g, carry):
                rows = g * _L + iota16
                iota8 = iota16 * 8
                accs = [jnp.zeros((_L,), jnp.float32)] * 8
                for j in range(D):
                    col = (iota8 + j) & (D - 1)
                    v = plsc.load_gather(emb_v, [rows, col]) + plsc.load_gather(
                        trans_v, [rows, col]
                    )
                    accs[j % 8] = accs[j % 8] + v * v
                tot = (accs[0] + accs[1]) + (accs[2] + accs[3])
                tot = tot + (accs[4] + accs[5]) + (accs[6] + accs[7])
                y = _rsqrt16(tot)
                inv = 1.0 / jnp.maximum(tot * y, 1e-12)
                for j in range(D):
                    col = (iota8 + j) & (D - 1)
                    v = plsc.load_gather(emb_v, [rows, col]) + plsc.load_gather(
                        trans_v, [rows, col]
                    )
                    plsc.store_scatter(out_v, [rows, col], v * inv)
                return carry

            lax.fori_loop(0, groups_per_chunk, group, 0)
            pltpu.sync_copy(out_v, out_hbm.at[pl.ds(row0, chunk)])

    return k


def kernel(embeddings, operator_idxs, entity_list, relation_dim, entity_dim, flag, rel_id, translations):
    B, D = embeddings.shape
    k = _make_sc_kernel(B, D, chunk=256)
    return k(translations, operator_idxs, embeddings)


# speedup vs baseline: 1.2439x; 1.2439x over previous
import functools

import jax
import jax.numpy as jnp
from jax import lax
from jax.experimental import pallas as pl
from jax.experimental.pallas import tpu as pltpu
from jax.experimental.pallas import tpu_sc as plsc

_NC = 2
_NS = 16
_NW = _NC * _NS
_L = 16


def _rsqrt16(s):
    i = lax.bitcast_convert_type(s, jnp.int32)
    i = jnp.int32(0x5F3759DF) - (i >> 1)
    y = lax.bitcast_convert_type(i, jnp.float32)
    for _ in range(3):
        y = y * (1.5 - 0.5 * s * y * y)
    return y


def _make_sc_kernel(B, D, chunk):
    b_per_w = B // _NW
    n_chunks = b_per_w // chunk
    n_idx_sub = chunk // 128
    groups_per_chunk = chunk // _L
    mesh = plsc.VectorSubcoreMesh(core_axis_name="c", subcore_axis_name="s")

    @functools.partial(
        pl.kernel,
        mesh=mesh,
        compiler_params=pltpu.CompilerParams(
            needs_layout_passes=False,
            use_tc_tiling_on_sc=False,
        ),
        out_type=jax.ShapeDtypeStruct((B, D), jnp.float32),
        scratch_types=[
            pltpu.VMEM((b_per_w // 128, 128), jnp.int32),
            pltpu.VMEM((2, chunk, D), jnp.float32),
            pltpu.VMEM((2, chunk, D), jnp.float32),
            pltpu.VMEM((2, chunk, D), jnp.float32),
            pltpu.SemaphoreType.DMA,
            pltpu.SemaphoreType.DMA,
        ],
    )
    def k(trans_hbm, idx_hbm, emb_hbm, out_hbm, idx_v, trans_v, emb_v, out_v,
          sem_in, sem_out):
        wid = lax.axis_index("s") * _NC + lax.axis_index("c")
        base = wid * b_per_w
        iota16 = lax.iota(jnp.int32, _L)

        for r in range(b_per_w // 128):
            pltpu.sync_copy(idx_hbm.at[pl.ds(base + r * 128, 128)], idx_v.at[r])

        def fire_in(c):
            p = c & 1
            cps = []
            for r in range(n_idx_sub):
                cps.append(
                    pltpu.async_copy(
                        trans_hbm.at[idx_v.at[c * n_idx_sub + r]],
                        trans_v.at[p].at[pl.ds(r * 128, 128)],
                        sem_in,
                    )
                )
            cps.append(
                pltpu.async_copy(
                    emb_hbm.at[pl.ds(base + c * chunk, chunk)], emb_v.at[p], sem_in
                )
            )
            return cps

        in_flight = fire_in(0)
        out_flight = [None, None]
        for c in range(n_chunks):
            p = c & 1
            for cp in in_flight:
                cp.wait()
            if c + 1 < n_chunks:
                in_flight = fire_in(c + 1)
            if out_flight[p] is not None:
                out_flight[p].wait()

            ev = emb_v.at[p]
            tv = trans_v.at[p]
            ov = out_v.at[p]

            def group(g, carry):
                rows = g * _L + iota16
                accs = [jnp.zeros((_L,), jnp.float32)] * 8
                for j in range(D):
                    col = (iota16 + j) & (D - 1)
                    v = plsc.load_gather(ev, [rows, col]) + plsc.load_gather(
                        tv, [rows, col]
                    )
                    plsc.store_scatter(ov, [rows, col], v)
                    accs[j % 8] = accs[j % 8] + v * v
                tot = (accs[0] + accs[1]) + (accs[2] + accs[3])
                tot = tot + (accs[4] + accs[5]) + (accs[6] + accs[7])
                y = _rsqrt16(tot)
                inv = 1.0 / jnp.maximum(tot * y, 1e-12)
                for j in range(D):
                    col = (iota16 + j) & (D - 1)
                    v = plsc.load_gather(ov, [rows, col])
                    plsc.store_scatter(ov, [rows, col], v * inv)
                return carry

            lax.fori_loop(0, groups_per_chunk, group, 0)
            out_flight[p] = pltpu.async_copy(
                ov, out_hbm.at[pl.ds(base + c * chunk, chunk)], sem_out
            )
        for cp in out_flight:
            if cp is not None:
                cp.wait()

    return k


def kernel(embeddings, operator_idxs, entity_list, relation_dim, entity_dim, flag, rel_id, translations):
    B, D = embeddings.shape
    k = _make_sc_kernel(B, D, chunk=128)
    return k(translations, operator_idxs, embeddings)


# speedup vs baseline: 1.2987x; 1.0441x over previous
import functools

import jax
import jax.numpy as jnp
from jax import lax
from jax.experimental import pallas as pl
from jax.experimental.pallas import tpu as pltpu
from jax.experimental.pallas import tpu_sc as plsc

_NC = 2
_NS = 16
_NW = _NC * _NS
_L = 16


def _rsqrt16(s):
    i = lax.bitcast_convert_type(s, jnp.int32)
    i = jnp.int32(0x5F3759DF) - (i >> 1)
    y = lax.bitcast_convert_type(i, jnp.float32)
    for _ in range(3):
        y = y * (1.5 - 0.5 * s * y * y)
    return y


def _make_sc_kernel(B, D, chunk):
    b_per_w = B // _NW
    n_chunks = b_per_w // chunk
    n_idx_sub = chunk // 128
    groups_per_chunk = chunk // _L
    mesh = plsc.VectorSubcoreMesh(core_axis_name="c", subcore_axis_name="s")

    @functools.partial(
        pl.kernel,
        mesh=mesh,
        compiler_params=pltpu.CompilerParams(
            needs_layout_passes=False,
            use_tc_tiling_on_sc=False,
        ),
        out_type=jax.ShapeDtypeStruct((B, D), jnp.float32),
        scratch_types=[
            pltpu.VMEM((b_per_w // 128, 128), jnp.int32),
            pltpu.VMEM((2, chunk, D), jnp.float32),
            pltpu.VMEM((2, chunk, D), jnp.float32),
            pltpu.VMEM((2, chunk, D), jnp.float32),
            pltpu.SemaphoreType.DMA,
            pltpu.SemaphoreType.DMA,
        ],
    )
    def k(trans_hbm, idx_hbm, emb_hbm, out_hbm, idx_v, trans_v, emb_v, out_v,
          sem_in, sem_out):
        wid = lax.axis_index("s") * _NC + lax.axis_index("c")
        base = wid * b_per_w
        iota16 = lax.iota(jnp.int32, _L)

        for r in range(b_per_w // 128):
            pltpu.sync_copy(idx_hbm.at[pl.ds(base + r * 128, 128)], idx_v.at[r])

        def fire_in(c):
            p = c & 1
            cps = []
            for r in range(n_idx_sub):
                cps.append(
                    pltpu.async_copy(
                        trans_hbm.at[idx_v.at[c * n_idx_sub + r]],
                        trans_v.at[p].at[pl.ds(r * 128, 128)],
                        sem_in,
                    )
                )
            cps.append(
                pltpu.async_copy(
                    emb_hbm.at[pl.ds(base + c * chunk, chunk)], emb_v.at[p], sem_in
                )
            )
            return cps

        in_flight = fire_in(0)
        out_flight = [None, None]
        for c in range(n_chunks):
            p = c & 1
            for cp in in_flight:
                cp.wait()
            if c + 1 < n_chunks:
                in_flight = fire_in(c + 1)
            if out_flight[p] is not None:
                out_flight[p].wait()

            ev = emb_v.at[p]
            tv = trans_v.at[p]
            ov = out_v.at[p]

            def group(g, carry):
                rows = g * _L + iota16
                accs = [jnp.zeros((_L,), jnp.float32)] * 8
                for j in range(D):
                    col = (iota16 + j) & (D - 1)
                    v = plsc.load_gather(ev, [rows, col]) + plsc.load_gather(
                        tv, [rows, col]
                    )
                    accs[j % 8] = accs[j % 8] + v * v
                tot = (accs[0] + accs[1]) + (accs[2] + accs[3])
                tot = tot + (accs[4] + accs[5]) + (accs[6] + accs[7])
                y = _rsqrt16(tot)
                inv = 1.0 / jnp.maximum(tot * y, 1e-12)
                for j in range(D):
                    col = (iota16 + j) & (D - 1)
                    v = plsc.load_gather(ev, [rows, col]) + plsc.load_gather(
                        tv, [rows, col]
                    )
                    plsc.store_scatter(ov, [rows, col], v * inv)
                return carry

            lax.fori_loop(0, groups_per_chunk, group, 0)
            out_flight[p] = pltpu.async_copy(
                ov, out_hbm.at[pl.ds(base + c * chunk, chunk)], sem_out
            )
        for cp in out_flight:
            if cp is not None:
                cp.wait()

    return k


def kernel(embeddings, operator_idxs, entity_list, relation_dim, entity_dim, flag, rel_id, translations):
    B, D = embeddings.shape
    k = _make_sc_kernel(B, D, chunk=128)
    return k(translations, operator_idxs, embeddings)


# speedup vs baseline: 1.5903x; 1.2246x over previous
import functools

import jax
import jax.numpy as jnp
from jax import lax
from jax.experimental import pallas as pl
from jax.experimental.pallas import tpu as pltpu
from jax.experimental.pallas import tpu_sc as plsc

_NC = 2
_NS = 16
_NW = _NC * _NS
_L = 16


def _rsqrt16(s):
    i = lax.bitcast_convert_type(s, jnp.int32)
    i = jnp.int32(0x5F3759DF) - (i >> 1)
    y = lax.bitcast_convert_type(i, jnp.float32)
    for _ in range(3):
        y = y * (1.5 - 0.5 * s * y * y)
    return y


def _make_sc_kernel(B, D, chunk):
    b_per_w = B // _NW
    n_chunks = b_per_w // chunk
    n_idx_sub = chunk // 128
    groups_per_chunk = chunk // _L
    n_sub = D // _L
    mesh = plsc.VectorSubcoreMesh(core_axis_name="c", subcore_axis_name="s")

    @functools.partial(
        pl.kernel,
        mesh=mesh,
        compiler_params=pltpu.CompilerParams(
            needs_layout_passes=False,
            use_tc_tiling_on_sc=False,
        ),
        out_type=jax.ShapeDtypeStruct((B, D), jnp.float32),
        scratch_types=[
            pltpu.VMEM((b_per_w // 128, 128), jnp.int32),
            pltpu.VMEM((chunk, D), jnp.float32),
            pltpu.VMEM((chunk, D), jnp.float32),
            pltpu.VMEM((chunk, D), jnp.float32),
            pltpu.VMEM((_L, _L), jnp.float32),
            pltpu.SemaphoreType.DMA,
        ],
    )
    def k(trans_hbm, idx_hbm, emb_hbm, out_hbm, idx_v, trans_v, emb_v, out_v,
          sums_v, sem):
        wid = lax.axis_index("s") * _NC + lax.axis_index("c")
        base = wid * b_per_w
        iota16 = lax.iota(jnp.int32, _L)

        for r in range(b_per_w // 128):
            pltpu.sync_copy(idx_hbm.at[pl.ds(base + r * 128, 128)], idx_v.at[r])

        for c in range(n_chunks):
            row0 = base + c * chunk
            copies = []
            for r in range(n_idx_sub):
                copies.append(
                    pltpu.async_copy(
                        trans_hbm.at[idx_v.at[c * n_idx_sub + r]],
                        trans_v.at[pl.ds(r * 128, 128)],
                        sem,
                    )
                )
            copies.append(pltpu.async_copy(emb_hbm.at[pl.ds(row0, chunk)], emb_v, sem))
            for cp in copies:
                cp.wait()

            def group(g, carry):
                grow = g * _L
                for r in range(_L):
                    row = grow + r
                    acc0 = acc1 = None
                    for gg in range(n_sub):
                        e = emb_v[row, pl.ds(gg * _L, _L)]
                        t = trans_v[row, pl.ds(gg * _L, _L)]
                        s = e + t
                        out_v[row, pl.ds(gg * _L, _L)] = s
                        if gg % 2 == 0:
                            acc0 = s * s if acc0 is None else acc0 + s * s
                        else:
                            acc1 = s * s if acc1 is None else acc1 + s * s
                    sums_v[r, :] = acc0 + acc1
                tots = [None] * 4
                for l in range(_L):
                    cl = (iota16 + l) & (_L - 1)
                    gcol = plsc.load_gather(sums_v, [iota16, cl])
                    q = l % 4
                    tots[q] = gcol if tots[q] is None else tots[q] + gcol
                tot = (tots[0] + tots[1]) + (tots[2] + tots[3])
                y = _rsqrt16(tot)
                inv = 1.0 / jnp.maximum(tot * y, 1e-12)
                rows = grow + iota16
                for j in range(D):
                    col = (iota16 + j) & (D - 1)
                    v = plsc.load_gather(out_v, [rows, col])
                    plsc.store_scatter(out_v, [rows, col], v * inv)
                return carry

            lax.fori_loop(0, groups_per_chunk, group, 0)
            pltpu.sync_copy(out_v, out_hbm.at[pl.ds(row0, chunk)])

    return k


def kernel(embeddings, operator_idxs, entity_list, relation_dim, entity_dim, flag, rel_id, translations):
    B, D = embeddings.shape
    k = _make_sc_kernel(B, D, chunk=256)
    return k(translations, operator_idxs, embeddings)


# speedup vs baseline: 2.2633x; 1.4232x over previous
import functools

import jax
import jax.numpy as jnp
from jax import lax
from jax.experimental import pallas as pl
from jax.experimental.pallas import tpu as pltpu
from jax.experimental.pallas import tpu_sc as plsc

_NC = 2
_NS = 16
_NW = _NC * _NS
_L = 16


def _rsqrt16(s):
    i = lax.bitcast_convert_type(s, jnp.int32)
    i = jnp.int32(0x5F3759DF) - (i >> 1)
    y = lax.bitcast_convert_type(i, jnp.float32)
    for _ in range(3):
        y = y * (1.5 - 0.5 * s * y * y)
    return y


def _make_sc_kernel(B, D, chunk):
    b_per_w = B // _NW
    n_chunks = b_per_w // chunk
    n_idx_sub = chunk // 128
    groups_per_chunk = chunk // _L
    n_sub = D // _L
    mesh = plsc.VectorSubcoreMesh(core_axis_name="c", subcore_axis_name="s")

    @functools.partial(
        pl.kernel,
        mesh=mesh,
        compiler_params=pltpu.CompilerParams(
            needs_layout_passes=False,
            use_tc_tiling_on_sc=False,
        ),
        out_type=jax.ShapeDtypeStruct((B, D), jnp.float32),
        scratch_types=[
            pltpu.VMEM((b_per_w // 128, 128), jnp.int32),
            pltpu.VMEM((chunk, D), jnp.float32),
            pltpu.VMEM((chunk, D), jnp.float32),
            pltpu.VMEM((chunk, D), jnp.float32),
            pltpu.VMEM((_L, _L), jnp.float32),
            pltpu.SemaphoreType.DMA,
        ],
    )
    def k(trans_hbm, idx_hbm, emb_hbm, out_hbm, idx_v, trans_v, emb_v, out_v,
          sums_v, sem):
        wid = lax.axis_index("s") * _NC + lax.axis_index("c")
        base = wid * b_per_w
        iota16 = lax.iota(jnp.int32, _L)

        for r in range(b_per_w // 128):
            pltpu.sync_copy(idx_hbm.at[pl.ds(base + r * 128, 128)], idx_v.at[r])

        for c in range(n_chunks):
            row0 = base + c * chunk
            copies = []
            for r in range(n_idx_sub):
                copies.append(
                    pltpu.async_copy(
                        trans_hbm.at[idx_v.at[c * n_idx_sub + r]],
                        trans_v.at[pl.ds(r * 128, 128)],
                        sem,
                    )
                )
            copies.append(pltpu.async_copy(emb_hbm.at[pl.ds(row0, chunk)], emb_v, sem))
            for cp in copies:
                cp.wait()

            def group(g, carry):
                grow = g * _L
                for r in range(_L):
                    row = grow + r
                    acc0 = acc1 = None
                    for gg in range(n_sub):
                        e = emb_v[row, pl.ds(gg * _L, _L)]
                        t = trans_v[row, pl.ds(gg * _L, _L)]
                        s = e + t
                        out_v[row, pl.ds(gg * _L, _L)] = s
                        if gg % 2 == 0:
                            acc0 = s * s if acc0 is None else acc0 + s * s
                        else:
                            acc1 = s * s if acc1 is None else acc1 + s * s
                    sums_v[r, :] = acc0 + acc1
                tots = [None] * 4
                for l in range(_L):
                    cl = (iota16 + l) & (_L - 1)
                    gcol = plsc.load_gather(sums_v, [iota16, cl])
                    q = l % 4
                    tots[q] = gcol if tots[q] is None else tots[q] + gcol
                tot = (tots[0] + tots[1]) + (tots[2] + tots[3])
                y = _rsqrt16(tot)
                inv = 1.0 / jnp.maximum(tot * y, 1e-12)
                for r in range(_L):
                    row = grow + r
                    ivr = inv[r]
                    for gg in range(n_sub):
                        s = out_v[row, pl.ds(gg * _L, _L)]
                        out_v[row, pl.ds(gg * _L, _L)] = s * ivr
                return carry

            lax.fori_loop(0, groups_per_chunk, group, 0)
            pltpu.sync_copy(out_v, out_hbm.at[pl.ds(row0, chunk)])

    return k


def kernel(embeddings, operator_idxs, entity_list, relation_dim, entity_dim, flag, rel_id, translations):
    B, D = embeddings.shape
    k = _make_sc_kernel(B, D, chunk=256)
    return k(translations, operator_idxs, embeddings)


# speedup vs baseline: 2.4575x; 1.0858x over previous
import functools

import jax
import jax.numpy as jnp
from jax import lax
from jax.experimental import pallas as pl
from jax.experimental.pallas import tpu as pltpu
from jax.experimental.pallas import tpu_sc as plsc

_NC = 2
_NS = 16
_NW = _NC * _NS
_L = 16


def _rsqrt16(s):
    i = lax.bitcast_convert_type(s, jnp.int32)
    i = jnp.int32(0x5F3759DF) - (i >> 1)
    y = lax.bitcast_convert_type(i, jnp.float32)
    for _ in range(3):
        y = y * (1.5 - 0.5 * s * y * y)
    return y


def _make_sc_kernel(B, D, chunk):
    b_per_w = B // _NW
    n_chunks = b_per_w // chunk
    n_idx_sub = chunk // 128
    groups_per_chunk = chunk // _L
    n_sub = D // _L
    mesh = plsc.VectorSubcoreMesh(core_axis_name="c", subcore_axis_name="s")

    @functools.partial(
        pl.kernel,
        mesh=mesh,
        compiler_params=pltpu.CompilerParams(
            needs_layout_passes=False,
            use_tc_tiling_on_sc=False,
        ),
        out_type=jax.ShapeDtypeStruct((B, D), jnp.float32),
        scratch_types=[
            pltpu.VMEM((b_per_w // 128, 128), jnp.int32),
            pltpu.VMEM((chunk, D), jnp.float32),
            pltpu.VMEM((chunk, D), jnp.float32),
            pltpu.VMEM((chunk, D), jnp.float32),
            pltpu.VMEM((chunk, D), jnp.float32),
            pltpu.VMEM((chunk, D), jnp.float32),
            pltpu.VMEM((chunk, D), jnp.float32),
            pltpu.VMEM((_L, _L), jnp.float32),
            pltpu.SemaphoreType.DMA,
            pltpu.SemaphoreType.DMA,
        ],
    )
    def k(trans_hbm, idx_hbm, emb_hbm, out_hbm, idx_v, emb0_v, emb1_v,
          trans0_v, trans1_v, out0_v, out1_v, sums_v, sem_in, sem_out):
        wid = lax.axis_index("s") * _NC + lax.axis_index("c")
        base = wid * b_per_w
        iota16 = lax.iota(jnp.int32, _L)
        embs = (emb0_v, emb1_v)
        transs = (trans0_v, trans1_v)
        outs = (out0_v, out1_v)

        for r in range(b_per_w // 128):
            pltpu.sync_copy(idx_hbm.at[pl.ds(base + r * 128, 128)], idx_v.at[r])

        def fire_in(c):
            p = c & 1
            cps = []
            for r in range(n_idx_sub):
                cps.append(
                    pltpu.async_copy(
                        trans_hbm.at[idx_v.at[c * n_idx_sub + r]],
                        transs[p].at[pl.ds(r * 128, 128)],
                        sem_in,
                    )
                )
            cps.append(
                pltpu.async_copy(
                    emb_hbm.at[pl.ds(base + c * chunk, chunk)], embs[p], sem_in
                )
            )
            return cps

        in_flight = fire_in(0)
        out_flight = [None, None]
        for c in range(n_chunks):
            p = c & 1
            emb_v, trans_v, out_v = embs[p], transs[p], outs[p]
            for cp in in_flight:
                cp.wait()
            if c + 1 < n_chunks:
                in_flight = fire_in(c + 1)
            if out_flight[p] is not None:
                out_flight[p].wait()

            def group(g, carry):
                grow = g * _L
                for r in range(_L):
                    row = grow + r
                    acc0 = acc1 = None
                    for gg in range(n_sub):
                        e = emb_v[row, pl.ds(gg * _L, _L)]
                        t = trans_v[row, pl.ds(gg * _L, _L)]
                        s = e + t
                        out_v[row, pl.ds(gg * _L, _L)] = s
                        if gg % 2 == 0:
                            acc0 = s * s if acc0 is None else acc0 + s * s
                        else:
                            acc1 = s * s if acc1 is None else acc1 + s * s
                    sums_v[r, :] = acc0 + acc1
                tots = [None] * 4
                for l in range(_L):
                    cl = (iota16 + l) & (_L - 1)
                    gcol = plsc.load_gather(sums_v, [iota16, cl])
                    q = l % 4
                    tots[q] = gcol if tots[q] is None else tots[q] + gcol
                tot = (tots[0] + tots[1]) + (tots[2] + tots[3])
                y = _rsqrt16(tot)
                inv = 1.0 / jnp.maximum(tot * y, 1e-12)
                for r in range(_L):
                    row = grow + r
                    ivr = inv[r]
                    for gg in range(n_sub):
                        s = out_v[row, pl.ds(gg * _L, _L)]
                        out_v[row, pl.ds(gg * _L, _L)] = s * ivr
                return carry

            lax.fori_loop(0, groups_per_chunk, group, 0)
            out_flight[p] = pltpu.async_copy(
                out_v, out_hbm.at[pl.ds(base + c * chunk, chunk)], sem_out
            )
        for cp in out_flight:
            if cp is not None:
                cp.wait()

    return k


def kernel(embeddings, operator_idxs, entity_list, relation_dim, entity_dim, flag, rel_id, translations):
    B, D = embeddings.shape
    k = _make_sc_kernel(B, D, chunk=128)
    return k(translations, operator_idxs, embeddings)
